# baseline (speedup 1.0000x reference)
import functools

import jax
import jax.numpy as jnp
from jax import lax
from jax.experimental import pallas as pl
from jax.experimental.pallas import tpu as pltpu
from jax.experimental.pallas import tpu_sc as plsc

N_DR = 10000; N_P = 10000; N_MF = 2000; N_BP = 4000; N_CC = 1000
H = 128; B = 4096
HH = H // 2
SB = 16
EPAD = 128 * 16 * SB


def _relu(x):
    return jnp.maximum(x, 0.0)


def _rup(x, m):
    return (x + m - 1) // m * m


def _segsum_body(graphs, *refs):
    G = len(graphs)
    tables = refs[0:G]
    srcs = refs[G:2 * G]
    dsts = refs[2 * G:3 * G]
    outs = refs[3 * G:4 * G]
    accs = refs[4 * G:5 * G]
    (srcbuf, dstbuf, rows0, rows1, zbuf,
     sg0, sg1, ss0, ss1) = refs[5 * G:]

    core = lax.axis_index("c")
    sub = lax.axis_index("s")

    zv = jnp.zeros((16,), jnp.float32)

    def zst(i, _):
        zbuf[i // 4, pl.ds((i % 4) * 16, 16)] = zv
        return 0

    lax.fori_loop(0, 8 * 4, zst, 0)
    for g, (n_src, npad, nrows) in enumerate(graphs):
        nr = npad // 16

        def zcp(j, _):
            pltpu.sync_copy(zbuf.at[pl.ds(0, 8), :],
                            accs[g].at[pl.ds(sub * nr + 8 * j, 8), :])
            return 0

        lax.fori_loop(0, nr // 8, zcp, 0)
    plsc.subcore_barrier()

    for g, (n_src, npad, nrows) in enumerate(graphs):
        table, acc = tables[g], accs[g]
        rpt = nrows // 16
        r0 = sub * rpt
        off = core * n_src

        def g_start(ch, rows, sem):
            pltpu.async_copy(table.at[srcbuf.at[ch]], rows, sem)

        def g_wait(ch, rows, sem):
            pltpu.make_async_copy(table.at[srcbuf.at[ch]], rows, sem).wait()

        def s_start(ch, rows, sem):
            pltpu.async_copy(rows, acc.at[dstbuf.at[ch]], sem, add=True)

        def s_wait(ch, rows, sem):
            pltpu.make_async_copy(rows, acc.at[dstbuf.at[ch]], sem).wait()

        def sblock(sb, _):
            pltpu.sync_copy(srcs[g].at[pl.ds(r0 + sb * SB, SB), :], srcbuf)
            pltpu.sync_copy(dsts[g].at[pl.ds(r0 + sb * SB, SB), :], dstbuf)

            def adj(i, _):
                r = i // 8
                c = (i % 8) * 16
                srcbuf[r, pl.ds(c, 16)] = srcbuf[r, pl.ds(c, 16)] + off
                return 0

            lax.fori_loop(0, SB * 8, adj, 0)
            g_start(0, rows0, sg0)

            def body2(j, _):
                ch0 = 2 * j
                ch1 = ch0 + 1
                g_wait(ch0, rows0, sg0)

                @pl.when(j > 0)
                def _():
                    s_wait(ch0, rows1, ss1)

                g_start(ch1, rows1, sg1)
                s_start(ch0, rows0, ss0)
                g_wait(ch1, rows1, sg1)
                s_wait(ch1, rows0, ss0)
                g_start(jnp.minimum(ch0 + 2, SB - 1), rows0, sg0)
                s_start(ch1, rows1, ss1)
                return 0

            lax.fori_loop(0, SB // 2, body2, 0)
            g_wait(0, rows0, sg0)
            s_wait(0, rows1, ss1)
            return 0

        lax.fori_loop(0, rpt // SB, sblock, 0)

    plsc.subcore_barrier()
    for g, (n_src, npad, nrows) in enumerate(graphs):
        nr = npad // 16
        pltpu.sync_copy(accs[g].at[pl.ds(sub * nr, nr), :],
                        outs[g].at[core, pl.ds(sub * nr, nr), :])


def _sc_segsum(specs):
    graphs = []
    tabs, srcs, dsts = [], [], []
    for table, src, dst, n_dst in specs:
        n_src = table.shape[0]
        e = src.shape[0]
        ep = _rup(e, EPAD)
        pad = ep - e
        npad = _rup(n_dst + 16, 128)
        if pad:
            fill = jnp.arange(pad, dtype=jnp.int32) % 16
            src = jnp.concatenate([src.astype(jnp.int32), fill])
            dst = jnp.concatenate([dst.astype(jnp.int32), (npad - 16) + fill])
        else:
            src = src.astype(jnp.int32)
            dst = dst.astype(jnp.int32)
        tab2 = jnp.concatenate([table[:, :HH], table[:, HH:]], axis=0)
        tabs.append(tab2)
        srcs.append(src.reshape(-1, 128))
        dsts.append(dst.reshape(-1, 128))
        graphs.append((n_src, npad, ep // 128))

    mesh = plsc.VectorSubcoreMesh(core_axis_name="c", subcore_axis_name="s")
    out_type = [jax.ShapeDtypeStruct((2, npad, HH), jnp.float32)
                for (_, npad, _) in graphs]
    scratch = ([pltpu.VMEM_SHARED((npad, HH), jnp.float32)
                for (_, npad, _) in graphs]
               + [pltpu.VMEM((SB, 128), jnp.int32),
                  pltpu.VMEM((SB, 128), jnp.int32),
                  pltpu.VMEM((128, HH), jnp.float32),
                  pltpu.VMEM((128, HH), jnp.float32),
                  pltpu.VMEM((8, HH), jnp.float32),
                  pltpu.SemaphoreType.DMA,
                  pltpu.SemaphoreType.DMA,
                  pltpu.SemaphoreType.DMA,
                  pltpu.SemaphoreType.DMA])
    k = pl.kernel(functools.partial(_segsum_body, graphs),
                  out_type=out_type, mesh=mesh, scratch_types=scratch,
                  compiler_params=pltpu.CompilerParams(
                      use_tc_tiling_on_sc=False))
    outs = k(*tabs, *srcs, *dsts)
    if not isinstance(outs, (list, tuple)):
        outs = [outs]
    res = []
    for o, (table, src, dst, n_dst) in zip(outs, specs):
        res.append(jnp.concatenate([o[0, :n_dst, :], o[1, :n_dst, :]], axis=1))
    return res


NROW = 10240
NROWD = NROW + 128
ESB = 2048
DPOS = NROW
ASLOT = NROWD
CSLOT = NROWD + 16


def _segmax_body(graphs, *refs):
    G = len(graphs)
    hts = refs[0:G]
    srcs = refs[G:2 * G]
    dsts = refs[2 * G:3 * G]
    outs = refs[3 * G:4 * G]
    (htb, acc, elect, sb0, db0, sb1, db1, si0, si1) = refs[4 * G:]

    core = lax.axis_index("c")
    sub = lax.axis_index("s")
    lanes = lax.iota(jnp.int32, 16)
    fz = jnp.zeros((16,), jnp.float32)
    iz = jnp.zeros((16,), jnp.int32)
    cslot = jnp.full((16,), CSLOT, jnp.int32)

    def popcnt(m):
        plsc.store_scatter(elect, [cslot], iz)
        plsc.addupdate_scatter(elect, [cslot], jnp.where(m, 1, 0))
        return plsc.load_gather(elect, [cslot])[0]

    def apply16(posd, s):
        for c in range(4):
            vals = plsc.load_gather(htb, [s + c * NROW])
            cur = plsc.load_gather(acc, [posd + c * NROWD])
            plsc.store_scatter(acc, [posd + c * NROWD],
                               jnp.maximum(cur, vals))

    def process(i, bs, bd):
        d = bd[pl.ds(i * 16, 16)]
        s = bs[pl.ds(i * 16, 16)]
        act = d >= 0
        dd = jnp.where(act, d, DPOS + lanes)
        plsc.store_scatter(elect, [dd], lanes)
        rb = plsc.load_gather(elect, [dd])
        win = act & (rb == lanes)
        apply16(jnp.where(win, dd, DPOS + lanes), s)
        lose = act & (~win)
        nl = popcnt(lose)

        @pl.when(nl > 0)
        def _():
            elect[pl.ds(ASLOT, 16)] = jnp.where(lose, 1, 0)

            def it(n):
                a = elect[pl.ds(ASLOT, 16)] > 0
                pos = jnp.where(a, dd, DPOS + lanes)
                plsc.store_scatter(elect, [pos], lanes)
                rb2 = plsc.load_gather(elect, [pos])
                w2 = a & (rb2 == lanes)
                apply16(jnp.where(w2, dd, DPOS + lanes), s)
                na = a & (~w2)
                elect[pl.ds(ASLOT, 16)] = jnp.where(na, 1, 0)
                return popcnt(na)

            lax.while_loop(lambda n: n > 0, it, nl)

    for g, (nsb, cg) in enumerate(graphs):
        @pl.when(core == cg)
        def _():
            for hp in range(2):
                blk = hp * 16 + sub
                pltpu.sync_copy(hts[g].at[blk], htb)

                def z(i, _):
                    acc[pl.ds(i * 16, 16)] = fz
                    return 0

                lax.fori_loop(0, 4 * NROWD // 16, z, 0)

                def ld(sb, bs, bd, sem):
                    pltpu.async_copy(srcs[g].at[pl.ds(sb * ESB, ESB)],
                                     bs, sem)
                    pltpu.async_copy(dsts[g].at[pl.ds(sb * ESB, ESB)],
                                     bd, sem)

                def ldw(sb, bs, bd, sem):
                    pltpu.make_async_copy(srcs[g].at[pl.ds(sb * ESB, ESB)],
                                          bs, sem).wait()
                    pltpu.make_async_copy(dsts[g].at[pl.ds(sb * ESB, ESB)],
                                          bd, sem).wait()

                def scan(bs, bd):
                    def sc(i, _):
                        process(i, bs, bd)
                        return 0

                    lax.fori_loop(0, ESB // 16, sc, 0)

                ld(0, sb0, db0, si0)
                ld(1, sb1, db1, si1)

                def sbpair(i, _):
                    s0 = 2 * i
                    s1 = s0 + 1
                    ldw(s0, sb0, db0, si0)
                    scan(sb0, db0)
                    ld(jnp.minimum(s0 + 2, nsb - 1), sb0, db0, si0)
                    ldw(s1, sb1, db1, si1)
                    scan(sb1, db1)
                    ld(jnp.minimum(s1 + 2, nsb - 1), sb1, db1, si1)
                    return 0

                lax.fori_loop(0, nsb // 2, sbpair, 0)
                ldw(0, sb0, db0, si0)
                ldw(0, sb1, db1, si1)
                for c in range(4):
                    pltpu.sync_copy(
                        acc.at[pl.ds(c * NROWD, NROW)],
                        outs[g].at[blk, pl.ds(c * NROW, NROW)])


def _sc_segmax(specs):
    graphs = []
    hts, srcs, dsts = [], [], []
    for table, src, dst, n_dst, cg in specs:
        e = src.shape[0]
        ep = _rup(e, 2 * ESB)
        pad = ep - e
        if pad:
            fill = jnp.arange(pad, dtype=jnp.int32) % 16
            src = jnp.concatenate([src.astype(jnp.int32), fill])
            dst = jnp.concatenate([dst.astype(jnp.int32),
                                   jnp.full((pad,), -1, jnp.int32)])
        else:
            src = src.astype(jnp.int32)
            dst = dst.astype(jnp.int32)
        tpad = jnp.pad(table, ((0, NROW - table.shape[0]), (0, 0)))
        hts.append(tpad.T.reshape(32, 4 * NROW))
        srcs.append(src)
        dsts.append(dst)
        graphs.append((ep // ESB, cg))

    mesh = plsc.VectorSubcoreMesh(core_axis_name="c", subcore_axis_name="s")
    out_type = [jax.ShapeDtypeStruct((32, 4 * NROW), jnp.float32)
                for _ in graphs]
    scratch = [pltpu.VMEM((4 * NROW,), jnp.float32),
               pltpu.VMEM((4 * NROWD,), jnp.float32),
               pltpu.VMEM((NROWD + 32,), jnp.int32),
               pltpu.VMEM((ESB,), jnp.int32),
               pltpu.VMEM((ESB,), jnp.int32),
               pltpu.VMEM((ESB,), jnp.int32),
               pltpu.VMEM((ESB,), jnp.int32),
               pltpu.SemaphoreType.DMA,
               pltpu.SemaphoreType.DMA]
    k = pl.kernel(functools.partial(_segmax_body, graphs),
                  out_type=out_type, mesh=mesh, scratch_types=scratch,
                  compiler_params=pltpu.CompilerParams(
                      needs_layout_passes=False))
    outs = k(*hts, *srcs, *dsts)
    if not isinstance(outs, (list, tuple)):
        outs = [outs]
    res = []
    for o, (table, src, dst, n_dst, cg) in zip(outs, specs):
        res.append(o.reshape(128, NROW)[:, :n_dst].T)
    return res


def _mlp_body(h_ref, w1, b1, g1, be1, w2, b2, g2, be2, w3, b3, g3, be3,
              wo, bo, out_ref):
    def bn_relu(x, g, b):
        mu = jnp.mean(x, axis=0, keepdims=True)
        var = jnp.mean((x - mu) ** 2, axis=0, keepdims=True)
        return _relu((x - mu) / jnp.sqrt(var + 1e-5) * g + b)

    x = h_ref[:]
    x = bn_relu(jnp.dot(x, w1[:], preferred_element_type=jnp.float32) + b1[0],
                g1[0], be1[0])
    x = bn_relu(jnp.dot(x, w2[:], preferred_element_type=jnp.float32) + b2[0],
                g2[0], be2[0])
    x = bn_relu(jnp.dot(x, w3[:], preferred_element_type=jnp.float32) + b3[0],
                g3[0], be3[0])
    out_ref[:] = jax.nn.sigmoid(
        jnp.dot(x, wo[:], preferred_element_type=jnp.float32) + bo[0])


def _mlp_head(h, p):
    args = [h]
    for nm in ["W1", "b1", "g1", "be1", "W2", "b2", "g2", "be2",
               "W3", "b3", "g3", "be3", "W_out", "b_out"]:
        v = p[nm]
        args.append(v.reshape(1, -1) if v.ndim == 1 else v)
    return pl.pallas_call(
        _mlp_body,
        out_shape=jax.ShapeDtypeStruct((B, 1), jnp.float32),
    )(*args)


def kernel(finger_feats, seq_feats, MF_feat, BP_feat, CC_feat, params,
           x_dr, x_p, ddi_ei, ppi_ei, mf_sim_ei, bp_sim_ei, cc_sim_ei,
           mf2p_ei, bp2p_ei, cc2p_ei):
    p = params
    h_dr_f = _relu(finger_feats @ p["W_dr_emb"] + p["b_dr_emb"])
    h_p_s = _relu(seq_feats @ p["W_p_emb"] + p["b_p_emb"])
    h_mf = _relu(p["W_mf_emb"] + p["b_mf_emb"])
    h_bp = _relu(p["W_bp_emb"] + p["b_bp_emb"])
    h_cc = _relu(p["W_cc_emb"] + p["b_cc_emb"])

    agg_mf, agg_bp, agg_cc = _sc_segsum([
        (h_mf, mf_sim_ei[0], mf_sim_ei[1], N_MF),
        (h_bp, bp_sim_ei[0], bp_sim_ei[1], N_BP),
        (h_cc, cc_sim_ei[0], cc_sim_ei[1], N_CC),
    ])
    mf_feat = _relu(agg_mf @ p["W_mf_sim"] + p["b_mf_sim"]) + h_mf
    bp_feat = _relu(agg_bp @ p["W_bp_sim"] + p["b_bp_sim"]) + h_bp
    cc_feat = _relu(agg_cc @ p["W_cc_sim"] + p["b_cc_sim"]) + h_cc

    g_mf = mf_feat @ p["W_mf2p"]
    g_bp = bp_feat @ p["W_bp2p"]
    g_cc = cc_feat @ p["W_cc2p"]
    agg_mf2p, agg_bp2p = _sc_segsum([
        (g_mf, mf2p_ei[0], mf2p_ei[1], N_P),
        (g_bp, bp2p_ei[0], bp2p_ei[1], N_P),
    ])
    (agg_cc2p,) = _sc_segsum([
        (g_cc, cc2p_ei[0], cc2p_ei[1], N_P),
    ])
    h_p_go = (_relu(agg_mf2p + p["b_mf2p"]) + _relu(agg_bp2p + p["b_bp2p"])
              + _relu(agg_cc2p + p["b_cc2p"]))

    pool_d1 = _relu(h_dr_f @ p["W_ddi_pool"] + p["b_ddi_pool"])
    pool_p1 = _relu(h_p_s @ p["W_ppi_pool"] + p["b_ppi_pool"])
    agg_d1, agg_p1 = _sc_segmax([
        (pool_d1, ddi_ei[0], ddi_ei[1], N_DR, 0),
        (pool_p1, ppi_ei[0], ppi_ei[1], N_P, 1),
    ])
    h_dr1 = _relu(h_dr_f @ p["W_ddi_self"] + agg_d1 @ p["W_ddi_neigh"]
                  + p["b_ddi"])
    h_p1 = _relu(h_p_s @ p["W_ppi_self"] + agg_p1 @ p["W_ppi_neigh"]
                 + p["b_ppi"])
    pool_d2 = _relu(h_dr1 @ p["W_ddi_pool"] + p["b_ddi_pool"])
    pool_p2 = _relu(h_p1 @ p["W_ppi_pool"] + p["b_ppi_pool"])
    agg_d2, agg_p2 = _sc_segmax([
        (pool_d2, ddi_ei[0], ddi_ei[1], N_DR, 0),
        (pool_p2, ppi_ei[0], ppi_ei[1], N_P, 1),
    ])
    h_dr2 = _relu(h_dr1 @ p["W_ddi_self"] + agg_d2 @ p["W_ddi_neigh"]
                  + p["b_ddi"])
    h_p2 = _relu(h_p1 @ p["W_ppi_self"] + agg_p2 @ p["W_ppi_neigh"]
                 + p["b_ppi"])

    dr_new = jnp.concatenate([h_dr_f, h_dr1, h_dr2], axis=1)
    p_new = jnp.concatenate([h_p_s, h_p1, h_p2, h_p_go], axis=1)
    h = jnp.concatenate([dr_new[x_dr[:, 0]], p_new[x_p[:, 0]]], axis=1)
    return _mlp_head(h, p)


# speedup vs baseline: 1.2115x; 1.2115x over previous
import functools

import jax
import jax.numpy as jnp
from jax import lax
from jax.experimental import pallas as pl
from jax.experimental.pallas import tpu as pltpu
from jax.experimental.pallas import tpu_sc as plsc

N_DR = 10000; N_P = 10000; N_MF = 2000; N_BP = 4000; N_CC = 1000
H = 128; B = 4096
HH = H // 2
SB = 16
EPAD = 128 * 16 * SB


def _relu(x):
    return jnp.maximum(x, 0.0)


def _rup(x, m):
    return (x + m - 1) // m * m


def _segsum_body(graphs, *refs):
    G = len(graphs)
    tables = refs[0:G]
    srcs = refs[G:2 * G]
    dsts = refs[2 * G:3 * G]
    outs = refs[3 * G:4 * G]
    accs = refs[4 * G:5 * G]
    (srcbuf, dstbuf, rows0, rows1, zbuf,
     sg0, sg1, ss0, ss1) = refs[5 * G:]

    core = lax.axis_index("c")
    sub = lax.axis_index("s")

    zv = jnp.zeros((16,), jnp.float32)

    def zst(i, _):
        zbuf[i // 4, pl.ds((i % 4) * 16, 16)] = zv
        return 0

    lax.fori_loop(0, 8 * 4, zst, 0)
    for g, (n_src, npad, nrows) in enumerate(graphs):
        nr = npad // 16

        def zcp(j, _):
            pltpu.sync_copy(zbuf.at[pl.ds(0, 8), :],
                            accs[g].at[pl.ds(sub * nr + 8 * j, 8), :])
            return 0

        lax.fori_loop(0, nr // 8, zcp, 0)
    plsc.subcore_barrier()

    for g, (n_src, npad, nrows) in enumerate(graphs):
        table, acc = tables[g], accs[g]
        rpt = nrows // 16
        r0 = sub * rpt
        off = core * n_src

        def g_start(ch, rows, sem):
            pltpu.async_copy(table.at[srcbuf.at[ch]], rows, sem)

        def g_wait(ch, rows, sem):
            pltpu.make_async_copy(table.at[srcbuf.at[ch]], rows, sem).wait()

        def s_start(ch, rows, sem):
            pltpu.async_copy(rows, acc.at[dstbuf.at[ch]], sem, add=True)

        def s_wait(ch, rows, sem):
            pltpu.make_async_copy(rows, acc.at[dstbuf.at[ch]], sem).wait()

        def sblock(sb, _):
            pltpu.sync_copy(srcs[g].at[pl.ds(r0 + sb * SB, SB), :], srcbuf)
            pltpu.sync_copy(dsts[g].at[pl.ds(r0 + sb * SB, SB), :], dstbuf)

            def adj(i, _):
                r = i // 8
                c = (i % 8) * 16
                srcbuf[r, pl.ds(c, 16)] = srcbuf[r, pl.ds(c, 16)] + off
                return 0

            lax.fori_loop(0, SB * 8, adj, 0)
            g_start(0, rows0, sg0)

            def body2(j, _):
                ch0 = 2 * j
                ch1 = ch0 + 1
                g_wait(ch0, rows0, sg0)

                @pl.when(j > 0)
                def _():
                    s_wait(ch0, rows1, ss1)

                g_start(ch1, rows1, sg1)
                s_start(ch0, rows0, ss0)
                g_wait(ch1, rows1, sg1)
                s_wait(ch1, rows0, ss0)
                g_start(jnp.minimum(ch0 + 2, SB - 1), rows0, sg0)
                s_start(ch1, rows1, ss1)
                return 0

            lax.fori_loop(0, SB // 2, body2, 0)
            g_wait(0, rows0, sg0)
            s_wait(0, rows1, ss1)
            return 0

        lax.fori_loop(0, rpt // SB, sblock, 0)

    plsc.subcore_barrier()
    for g, (n_src, npad, nrows) in enumerate(graphs):
        nr = npad // 16
        pltpu.sync_copy(accs[g].at[pl.ds(sub * nr, nr), :],
                        outs[g].at[core, pl.ds(sub * nr, nr), :])


def _sc_segsum(specs):
    graphs = []
    tabs, srcs, dsts = [], [], []
    for table, src, dst, n_dst in specs:
        n_src = table.shape[0]
        e = src.shape[0]
        ep = _rup(e, EPAD)
        pad = ep - e
        npad = _rup(n_dst + 16, 128)
        if pad:
            fill = jnp.arange(pad, dtype=jnp.int32) % 16
            src = jnp.concatenate([src.astype(jnp.int32), fill])
            dst = jnp.concatenate([dst.astype(jnp.int32), (npad - 16) + fill])
        else:
            src = src.astype(jnp.int32)
            dst = dst.astype(jnp.int32)
        tab2 = jnp.concatenate([table[:, :HH], table[:, HH:]], axis=0)
        tabs.append(tab2)
        srcs.append(src.reshape(-1, 128))
        dsts.append(dst.reshape(-1, 128))
        graphs.append((n_src, npad, ep // 128))

    mesh = plsc.VectorSubcoreMesh(core_axis_name="c", subcore_axis_name="s")
    out_type = [jax.ShapeDtypeStruct((2, npad, HH), jnp.float32)
                for (_, npad, _) in graphs]
    scratch = ([pltpu.VMEM_SHARED((npad, HH), jnp.float32)
                for (_, npad, _) in graphs]
               + [pltpu.VMEM((SB, 128), jnp.int32),
                  pltpu.VMEM((SB, 128), jnp.int32),
                  pltpu.VMEM((128, HH), jnp.float32),
                  pltpu.VMEM((128, HH), jnp.float32),
                  pltpu.VMEM((8, HH), jnp.float32),
                  pltpu.SemaphoreType.DMA,
                  pltpu.SemaphoreType.DMA,
                  pltpu.SemaphoreType.DMA,
                  pltpu.SemaphoreType.DMA])
    k = pl.kernel(functools.partial(_segsum_body, graphs),
                  out_type=out_type, mesh=mesh, scratch_types=scratch,
                  compiler_params=pltpu.CompilerParams(
                      use_tc_tiling_on_sc=False))
    outs = k(*tabs, *srcs, *dsts)
    if not isinstance(outs, (list, tuple)):
        outs = [outs]
    res = []
    for o, (table, src, dst, n_dst) in zip(outs, specs):
        res.append(jnp.concatenate([o[0, :n_dst, :], o[1, :n_dst, :]], axis=1))
    return res


NROW = 10240
NROWD = NROW + 128
ESB = 2048
DPOS = NROW
ASLOT = NROWD
CSLOT = NROWD + 16


def _segmax_body(graphs, *refs):
    G = len(graphs)
    hts = refs[0:G]
    srcs = refs[G:2 * G]
    dsts = refs[2 * G:3 * G]
    outs = refs[3 * G:4 * G]
    (htb, acc, elect, sb0, db0, sb1, db1, si0, si1) = refs[4 * G:]

    core = lax.axis_index("c")
    sub = lax.axis_index("s")
    lanes = lax.iota(jnp.int32, 16)
    fz = jnp.zeros((16,), jnp.float32)
    iz = jnp.zeros((16,), jnp.int32)
    cslot = jnp.full((16,), CSLOT, jnp.int32)

    def popcnt(m):
        plsc.store_scatter(elect, [cslot], iz)
        plsc.addupdate_scatter(elect, [cslot], jnp.where(m, 1, 0))
        return plsc.load_gather(elect, [cslot])[0]

    def apply16(posd, s):
        for c in range(4):
            vals = plsc.load_gather(htb, [s + c * NROW])
            cur = plsc.load_gather(acc, [posd + c * NROWD])
            plsc.store_scatter(acc, [posd + c * NROWD],
                               jnp.maximum(cur, vals))

    def process(i, bs, bd):
        d = bd[pl.ds(i * 16, 16)]
        s = bs[pl.ds(i * 16, 16)]
        act = d >= 0
        dd = jnp.where(act, d, DPOS + lanes)
        plsc.store_scatter(elect, [dd], lanes)
        rb = plsc.load_gather(elect, [dd])
        win = act & (rb == lanes)
        apply16(jnp.where(win, dd, DPOS + lanes), s)
        lose = act & (~win)
        nl = popcnt(lose)

        @pl.when(nl > 0)
        def _():
            elect[pl.ds(ASLOT, 16)] = jnp.where(lose, 1, 0)

            def it(n):
                a = elect[pl.ds(ASLOT, 16)] > 0
                pos = jnp.where(a, dd, DPOS + lanes)
                plsc.store_scatter(elect, [pos], lanes)
                rb2 = plsc.load_gather(elect, [pos])
                w2 = a & (rb2 == lanes)
                apply16(jnp.where(w2, dd, DPOS + lanes), s)
                na = a & (~w2)
                elect[pl.ds(ASLOT, 16)] = jnp.where(na, 1, 0)
                return popcnt(na)

            lax.while_loop(lambda n: n > 0, it, nl)

    for g, (nsb, cg) in enumerate(graphs):
        @pl.when(core == cg)
        def _():
            for hp in range(2):
                blk = hp * 16 + sub
                pltpu.sync_copy(hts[g].at[blk], htb)

                def z(i, _):
                    acc[pl.ds(i * 16, 16)] = fz
                    return 0

                lax.fori_loop(0, 4 * NROWD // 16, z, 0)

                def ld(sb, bs, bd, sem):
                    pltpu.async_copy(srcs[g].at[pl.ds(sb * ESB, ESB)],
                                     bs, sem)
                    pltpu.async_copy(dsts[g].at[pl.ds(sb * ESB, ESB)],
                                     bd, sem)

                def ldw(sb, bs, bd, sem):
                    pltpu.make_async_copy(srcs[g].at[pl.ds(sb * ESB, ESB)],
                                          bs, sem).wait()
                    pltpu.make_async_copy(dsts[g].at[pl.ds(sb * ESB, ESB)],
                                          bd, sem).wait()

                def scan(bs, bd):
                    def sc(i, _):
                        process(i, bs, bd)
                        return 0

                    lax.fori_loop(0, ESB // 16, sc, 0)

                ld(0, sb0, db0, si0)
                ld(1, sb1, db1, si1)

                def sbpair(i, _):
                    s0 = 2 * i
                    s1 = s0 + 1
                    ldw(s0, sb0, db0, si0)
                    scan(sb0, db0)
                    ld(jnp.minimum(s0 + 2, nsb - 1), sb0, db0, si0)
                    ldw(s1, sb1, db1, si1)
                    scan(sb1, db1)
                    ld(jnp.minimum(s1 + 2, nsb - 1), sb1, db1, si1)
                    return 0

                lax.fori_loop(0, nsb // 2, sbpair, 0)
                ldw(0, sb0, db0, si0)
                ldw(0, sb1, db1, si1)
                for c in range(4):
                    pltpu.sync_copy(
                        acc.at[pl.ds(c * NROWD, NROW)],
                        outs[g].at[blk, pl.ds(c * NROW, NROW)])


def _sc_segmax(specs):
    graphs = []
    hts, srcs, dsts = [], [], []
    for table, src, dst, n_dst, cg in specs:
        e = src.shape[0]
        ep = _rup(e, 2 * ESB)
        pad = ep - e
        if pad:
            fill = jnp.arange(pad, dtype=jnp.int32) % 16
            src = jnp.concatenate([src.astype(jnp.int32), fill])
            dst = jnp.concatenate([dst.astype(jnp.int32),
                                   jnp.full((pad,), -1, jnp.int32)])
        else:
            src = src.astype(jnp.int32)
            dst = dst.astype(jnp.int32)
        tpad = jnp.pad(table, ((0, NROW - table.shape[0]), (0, 0)))
        hts.append(tpad.T.reshape(32, 4 * NROW))
        srcs.append(src)
        dsts.append(dst)
        graphs.append((ep // ESB, cg))

    mesh = plsc.VectorSubcoreMesh(core_axis_name="c", subcore_axis_name="s")
    out_type = [jax.ShapeDtypeStruct((32, 4 * NROW), jnp.float32)
                for _ in graphs]
    scratch = [pltpu.VMEM((4 * NROW,), jnp.float32),
               pltpu.VMEM((4 * NROWD,), jnp.float32),
               pltpu.VMEM((NROWD + 32,), jnp.int32),
               pltpu.VMEM((ESB,), jnp.int32),
               pltpu.VMEM((ESB,), jnp.int32),
               pltpu.VMEM((ESB,), jnp.int32),
               pltpu.VMEM((ESB,), jnp.int32),
               pltpu.SemaphoreType.DMA,
               pltpu.SemaphoreType.DMA]
    k = pl.kernel(functools.partial(_segmax_body, graphs),
                  out_type=out_type, mesh=mesh, scratch_types=scratch,
                  compiler_params=pltpu.CompilerParams(
                      needs_layout_passes=False))
    outs = k(*hts, *srcs, *dsts)
    if not isinstance(outs, (list, tuple)):
        outs = [outs]
    res = []
    for o, (table, src, dst, n_dst, cg) in zip(outs, specs):
        res.append(o.reshape(128, NROW)[:, :n_dst].T)
    return res


def _mlp_body(h_ref, w1, b1, g1, be1, w2, b2, g2, be2, w3, b3, g3, be3,
              wo, bo, out_ref):
    def bn_relu(x, g, b):
        mu = jnp.mean(x, axis=0, keepdims=True)
        var = jnp.mean((x - mu) ** 2, axis=0, keepdims=True)
        return _relu((x - mu) / jnp.sqrt(var + 1e-5) * g + b)

    x = h_ref[:]
    x = bn_relu(jnp.dot(x, w1[:], preferred_element_type=jnp.float32) + b1[0],
                g1[0], be1[0])
    x = bn_relu(jnp.dot(x, w2[:], preferred_element_type=jnp.float32) + b2[0],
                g2[0], be2[0])
    x = bn_relu(jnp.dot(x, w3[:], preferred_element_type=jnp.float32) + b3[0],
                g3[0], be3[0])
    out_ref[:] = jax.nn.sigmoid(
        jnp.dot(x, wo[:], preferred_element_type=jnp.float32) + bo[0])


def _mlp_head(h, p):
    args = [h]
    for nm in ["W1", "b1", "g1", "be1", "W2", "b2", "g2", "be2",
               "W3", "b3", "g3", "be3", "W_out", "b_out"]:
        v = p[nm]
        args.append(v.reshape(1, -1) if v.ndim == 1 else v)
    return pl.pallas_call(
        _mlp_body,
        out_shape=jax.ShapeDtypeStruct((B, 1), jnp.float32),
    )(*args)


def _segmax_ref(feat, ei, n):
    agg = jax.ops.segment_max(feat[ei[0]], ei[1], num_segments=n)
    return jnp.where(jnp.isfinite(agg), agg, 0.0)


def kernel(finger_feats, seq_feats, MF_feat, BP_feat, CC_feat, params,
           x_dr, x_p, ddi_ei, ppi_ei, mf_sim_ei, bp_sim_ei, cc_sim_ei,
           mf2p_ei, bp2p_ei, cc2p_ei):
    p = params
    h_dr_f = _relu(finger_feats @ p["W_dr_emb"] + p["b_dr_emb"])
    h_p_s = _relu(seq_feats @ p["W_p_emb"] + p["b_p_emb"])
    h_mf = _relu(p["W_mf_emb"] + p["b_mf_emb"])
    h_bp = _relu(p["W_bp_emb"] + p["b_bp_emb"])
    h_cc = _relu(p["W_cc_emb"] + p["b_cc_emb"])

    agg_mf, agg_bp, agg_cc = _sc_segsum([
        (h_mf, mf_sim_ei[0], mf_sim_ei[1], N_MF),
        (h_bp, bp_sim_ei[0], bp_sim_ei[1], N_BP),
        (h_cc, cc_sim_ei[0], cc_sim_ei[1], N_CC),
    ])
    mf_feat = _relu(agg_mf @ p["W_mf_sim"] + p["b_mf_sim"]) + h_mf
    bp_feat = _relu(agg_bp @ p["W_bp_sim"] + p["b_bp_sim"]) + h_bp
    cc_feat = _relu(agg_cc @ p["W_cc_sim"] + p["b_cc_sim"]) + h_cc

    g_mf = mf_feat @ p["W_mf2p"]
    g_bp = bp_feat @ p["W_bp2p"]
    g_cc = cc_feat @ p["W_cc2p"]
    agg_mf2p, agg_bp2p = _sc_segsum([
        (g_mf, mf2p_ei[0], mf2p_ei[1], N_P),
        (g_bp, bp2p_ei[0], bp2p_ei[1], N_P),
    ])
    (agg_cc2p,) = _sc_segsum([
        (g_cc, cc2p_ei[0], cc2p_ei[1], N_P),
    ])
    h_p_go = (_relu(agg_mf2p + p["b_mf2p"]) + _relu(agg_bp2p + p["b_bp2p"])
              + _relu(agg_cc2p + p["b_cc2p"]))

    pool_d1 = _relu(h_dr_f @ p["W_ddi_pool"] + p["b_ddi_pool"])
    pool_p1 = _relu(h_p_s @ p["W_ppi_pool"] + p["b_ppi_pool"])
    agg_d1 = _segmax_ref(pool_d1, ddi_ei, N_DR)
    agg_p1 = _segmax_ref(pool_p1, ppi_ei, N_P)
    h_dr1 = _relu(h_dr_f @ p["W_ddi_self"] + agg_d1 @ p["W_ddi_neigh"]
                  + p["b_ddi"])
    h_p1 = _relu(h_p_s @ p["W_ppi_self"] + agg_p1 @ p["W_ppi_neigh"]
                 + p["b_ppi"])
    pool_d2 = _relu(h_dr1 @ p["W_ddi_pool"] + p["b_ddi_pool"])
    pool_p2 = _relu(h_p1 @ p["W_ppi_pool"] + p["b_ppi_pool"])
    agg_d2 = _segmax_ref(pool_d2, ddi_ei, N_DR)
    agg_p2 = _segmax_ref(pool_p2, ppi_ei, N_P)
    h_dr2 = _relu(h_dr1 @ p["W_ddi_self"] + agg_d2 @ p["W_ddi_neigh"]
                  + p["b_ddi"])
    h_p2 = _relu(h_p1 @ p["W_ppi_self"] + agg_p2 @ p["W_ppi_neigh"]
                 + p["b_ppi"])

    dr_new = jnp.concatenate([h_dr_f, h_dr1, h_dr2], axis=1)
    p_new = jnp.concatenate([h_p_s, h_p1, h_p2, h_p_go], axis=1)
    h = jnp.concatenate([dr_new[x_dr[:, 0]], p_new[x_p[:, 0]]], axis=1)
    return _mlp_head(h, p)


# speedup vs baseline: 1.2158x; 1.0035x over previous
import functools

import jax
import jax.numpy as jnp
from jax import lax
from jax.experimental import pallas as pl
from jax.experimental.pallas import tpu as pltpu
from jax.experimental.pallas import tpu_sc as plsc

N_DR = 10000; N_P = 10000; N_MF = 2000; N_BP = 4000; N_CC = 1000
H = 128; B = 4096
HH = H // 2
SB = 16
EPAD = 128 * 16 * SB


def _relu(x):
    return jnp.maximum(x, 0.0)


def _rup(x, m):
    return (x + m - 1) // m * m


def _segsum_body(graphs, *refs):
    G = len(graphs)
    tables = refs[0:G]
    srcs = refs[G:2 * G]
    dsts = refs[2 * G:3 * G]
    outs = refs[3 * G:4 * G]
    accs = refs[4 * G:5 * G]
    (srcbuf, dstbuf, rows0, rows1, zbuf,
     sg0, sg1, ss0, ss1) = refs[5 * G:]

    core = lax.axis_index("c")
    sub = lax.axis_index("s")

    zv = jnp.zeros((16,), jnp.float32)

    def zst(i, _):
        zbuf[i // 4, pl.ds((i % 4) * 16, 16)] = zv
        return 0

    lax.fori_loop(0, 8 * 4, zst, 0)
    for g, (n_src, npad, nrows) in enumerate(graphs):
        nr = npad // 16

        def zcp(j, _):
            pltpu.sync_copy(zbuf.at[pl.ds(0, 8), :],
                            accs[g].at[pl.ds(sub * nr + 8 * j, 8), :])
            return 0

        lax.fori_loop(0, nr // 8, zcp, 0)
    plsc.subcore_barrier()

    for g, (n_src, npad, nrows) in enumerate(graphs):
        table, acc = tables[g], accs[g]
        rpt = nrows // 16
        r0 = sub * rpt
        off = core * n_src

        def g_start(ch, rows, sem):
            pltpu.async_copy(table.at[srcbuf.at[ch]], rows, sem)

        def g_wait(ch, rows, sem):
            pltpu.make_async_copy(table.at[srcbuf.at[ch]], rows, sem).wait()

        def s_start(ch, rows, sem):
            pltpu.async_copy(rows, acc.at[dstbuf.at[ch]], sem, add=True)

        def s_wait(ch, rows, sem):
            pltpu.make_async_copy(rows, acc.at[dstbuf.at[ch]], sem).wait()

        def sblock(sb, _):
            pltpu.sync_copy(srcs[g].at[pl.ds(r0 + sb * SB, SB), :], srcbuf)
            pltpu.sync_copy(dsts[g].at[pl.ds(r0 + sb * SB, SB), :], dstbuf)

            def adj(i, _):
                r = i // 8
                c = (i % 8) * 16
                srcbuf[r, pl.ds(c, 16)] = srcbuf[r, pl.ds(c, 16)] + off
                return 0

            lax.fori_loop(0, SB * 8, adj, 0)
            g_start(0, rows0, sg0)

            def body2(j, _):
                ch0 = 2 * j
                ch1 = ch0 + 1
                g_wait(ch0, rows0, sg0)

                @pl.when(j > 0)
                def _():
                    s_wait(ch0, rows1, ss1)

                g_start(ch1, rows1, sg1)
                s_start(ch0, rows0, ss0)
                g_wait(ch1, rows1, sg1)
                s_wait(ch1, rows0, ss0)
                g_start(jnp.minimum(ch0 + 2, SB - 1), rows0, sg0)
                s_start(ch1, rows1, ss1)
                return 0

            lax.fori_loop(0, SB // 2, body2, 0)
            g_wait(0, rows0, sg0)
            s_wait(0, rows1, ss1)
            return 0

        lax.fori_loop(0, rpt // SB, sblock, 0)

    plsc.subcore_barrier()
    for g, (n_src, npad, nrows) in enumerate(graphs):
        nr = npad // 16
        pltpu.sync_copy(accs[g].at[pl.ds(sub * nr, nr), :],
                        outs[g].at[core, pl.ds(sub * nr, nr), :])


def _sc_segsum(specs):
    graphs = []
    tabs, srcs, dsts = [], [], []
    for table, src, dst, n_dst in specs:
        n_src = table.shape[0]
        e = src.shape[0]
        ep = _rup(e, EPAD)
        pad = ep - e
        npad = _rup(n_dst + 16, 128)
        if pad:
            fill = jnp.arange(pad, dtype=jnp.int32) % 16
            src = jnp.concatenate([src.astype(jnp.int32), fill])
            dst = jnp.concatenate([dst.astype(jnp.int32), (npad - 16) + fill])
        else:
            src = src.astype(jnp.int32)
            dst = dst.astype(jnp.int32)
        tab2 = jnp.concatenate([table[:, :HH], table[:, HH:]], axis=0)
        tabs.append(tab2)
        srcs.append(src.reshape(-1, 128))
        dsts.append(dst.reshape(-1, 128))
        graphs.append((n_src, npad, ep // 128))

    mesh = plsc.VectorSubcoreMesh(core_axis_name="c", subcore_axis_name="s")
    out_type = [jax.ShapeDtypeStruct((2, npad, HH), jnp.float32)
                for (_, npad, _) in graphs]
    scratch = ([pltpu.VMEM_SHARED((npad, HH), jnp.float32)
                for (_, npad, _) in graphs]
               + [pltpu.VMEM((SB, 128), jnp.int32),
                  pltpu.VMEM((SB, 128), jnp.int32),
                  pltpu.VMEM((128, HH), jnp.float32),
                  pltpu.VMEM((128, HH), jnp.float32),
                  pltpu.VMEM((8, HH), jnp.float32),
                  pltpu.SemaphoreType.DMA,
                  pltpu.SemaphoreType.DMA,
                  pltpu.SemaphoreType.DMA,
                  pltpu.SemaphoreType.DMA])
    k = pl.kernel(functools.partial(_segsum_body, graphs),
                  out_type=out_type, mesh=mesh, scratch_types=scratch,
                  compiler_params=pltpu.CompilerParams(
                      use_tc_tiling_on_sc=False))
    outs = k(*tabs, *srcs, *dsts)
    if not isinstance(outs, (list, tuple)):
        outs = [outs]
    res = []
    for o, (table, src, dst, n_dst) in zip(outs, specs):
        res.append(jnp.concatenate([o[0, :n_dst, :], o[1, :n_dst, :]], axis=1))
    return res


NROW = 10240
NROWD = NROW + 128
ESB = 2048
DPOS = NROW
ASLOT = NROWD
CSLOT = NROWD + 16


def _segmax_body(graphs, *refs):
    G = len(graphs)
    hts = refs[0:G]
    srcs = refs[G:2 * G]
    dsts = refs[2 * G:3 * G]
    outs = refs[3 * G:4 * G]
    (htb, acc, elect, sb0, db0, sb1, db1, si0, si1) = refs[4 * G:]

    core = lax.axis_index("c")
    sub = lax.axis_index("s")
    lanes = lax.iota(jnp.int32, 16)
    fz = jnp.zeros((16,), jnp.float32)
    iz = jnp.zeros((16,), jnp.int32)
    cslot = jnp.full((16,), CSLOT, jnp.int32)

    def popcnt(m):
        plsc.store_scatter(elect, [cslot], iz)
        plsc.addupdate_scatter(elect, [cslot], jnp.where(m, 1, 0))
        return plsc.load_gather(elect, [cslot])[0]

    def apply16(posd, s):
        for c in range(4):
            vals = plsc.load_gather(htb, [s + c * NROW])
            cur = plsc.load_gather(acc, [posd + c * NROWD])
            plsc.store_scatter(acc, [posd + c * NROWD],
                               jnp.maximum(cur, vals))

    def process(i, bs, bd):
        d = bd[pl.ds(i * 16, 16)]
        s = bs[pl.ds(i * 16, 16)]
        act = d >= 0
        dd = jnp.where(act, d, DPOS + lanes)
        plsc.store_scatter(elect, [dd], lanes)
        rb = plsc.load_gather(elect, [dd])
        win = act & (rb == lanes)
        apply16(jnp.where(win, dd, DPOS + lanes), s)
        lose = act & (~win)
        nl = popcnt(lose)

        @pl.when(nl > 0)
        def _():
            elect[pl.ds(ASLOT, 16)] = jnp.where(lose, 1, 0)

            def it(n):
                a = elect[pl.ds(ASLOT, 16)] > 0
                pos = jnp.where(a, dd, DPOS + lanes)
                plsc.store_scatter(elect, [pos], lanes)
                rb2 = plsc.load_gather(elect, [pos])
                w2 = a & (rb2 == lanes)
                apply16(jnp.where(w2, dd, DPOS + lanes), s)
                na = a & (~w2)
                elect[pl.ds(ASLOT, 16)] = jnp.where(na, 1, 0)
                return popcnt(na)

            lax.while_loop(lambda n: n > 0, it, nl)

    for g, (nsb, cg) in enumerate(graphs):
        @pl.when(core == cg)
        def _():
            for hp in range(2):
                blk = hp * 16 + sub
                pltpu.sync_copy(hts[g].at[blk], htb)

                def z(i, _):
                    acc[pl.ds(i * 16, 16)] = fz
                    return 0

                lax.fori_loop(0, 4 * NROWD // 16, z, 0)

                def ld(sb, bs, bd, sem):
                    pltpu.async_copy(srcs[g].at[pl.ds(sb * ESB, ESB)],
                                     bs, sem)
                    pltpu.async_copy(dsts[g].at[pl.ds(sb * ESB, ESB)],
                                     bd, sem)

                def ldw(sb, bs, bd, sem):
                    pltpu.make_async_copy(srcs[g].at[pl.ds(sb * ESB, ESB)],
                                          bs, sem).wait()
                    pltpu.make_async_copy(dsts[g].at[pl.ds(sb * ESB, ESB)],
                                          bd, sem).wait()

                def scan(bs, bd):
                    def sc(i, _):
                        process(i, bs, bd)
                        return 0

                    lax.fori_loop(0, ESB // 16, sc, 0)

                ld(0, sb0, db0, si0)
                ld(1, sb1, db1, si1)

                def sbpair(i, _):
                    s0 = 2 * i
                    s1 = s0 + 1
                    ldw(s0, sb0, db0, si0)
                    scan(sb0, db0)
                    ld(jnp.minimum(s0 + 2, nsb - 1), sb0, db0, si0)
                    ldw(s1, sb1, db1, si1)
                    scan(sb1, db1)
                    ld(jnp.minimum(s1 + 2, nsb - 1), sb1, db1, si1)
                    return 0

                lax.fori_loop(0, nsb // 2, sbpair, 0)
                ldw(0, sb0, db0, si0)
                ldw(0, sb1, db1, si1)
                for c in range(4):
                    pltpu.sync_copy(
                        acc.at[pl.ds(c * NROWD, NROW)],
                        outs[g].at[blk, pl.ds(c * NROW, NROW)])


def _sc_segmax(specs):
    graphs = []
    hts, srcs, dsts = [], [], []
    for table, src, dst, n_dst, cg in specs:
        e = src.shape[0]
        ep = _rup(e, 2 * ESB)
        pad = ep - e
        if pad:
            fill = jnp.arange(pad, dtype=jnp.int32) % 16
            src = jnp.concatenate([src.astype(jnp.int32), fill])
            dst = jnp.concatenate([dst.astype(jnp.int32),
                                   jnp.full((pad,), -1, jnp.int32)])
        else:
            src = src.astype(jnp.int32)
            dst = dst.astype(jnp.int32)
        tpad = jnp.pad(table, ((0, NROW - table.shape[0]), (0, 0)))
        hts.append(tpad.T.reshape(32, 4 * NROW))
        srcs.append(src)
        dsts.append(dst)
        graphs.append((ep // ESB, cg))

    mesh = plsc.VectorSubcoreMesh(core_axis_name="c", subcore_axis_name="s")
    out_type = [jax.ShapeDtypeStruct((32, 4 * NROW), jnp.float32)
                for _ in graphs]
    scratch = [pltpu.VMEM((4 * NROW,), jnp.float32),
               pltpu.VMEM((4 * NROWD,), jnp.float32),
               pltpu.VMEM((NROWD + 32,), jnp.int32),
               pltpu.VMEM((ESB,), jnp.int32),
               pltpu.VMEM((ESB,), jnp.int32),
               pltpu.VMEM((ESB,), jnp.int32),
               pltpu.VMEM((ESB,), jnp.int32),
               pltpu.SemaphoreType.DMA,
               pltpu.SemaphoreType.DMA]
    k = pl.kernel(functools.partial(_segmax_body, graphs),
                  out_type=out_type, mesh=mesh, scratch_types=scratch,
                  compiler_params=pltpu.CompilerParams(
                      needs_layout_passes=False))
    outs = k(*hts, *srcs, *dsts)
    if not isinstance(outs, (list, tuple)):
        outs = [outs]
    res = []
    for o, (table, src, dst, n_dst, cg) in zip(outs, specs):
        res.append(o.reshape(128, NROW)[:, :n_dst].T)
    return res


def _mlp_body(h_ref, w1, b1, g1, be1, w2, b2, g2, be2, w3, b3, g3, be3,
              wo, bo, out_ref):
    def bn_relu(x, g, b):
        mu = jnp.mean(x, axis=0, keepdims=True)
        var = jnp.mean((x - mu) ** 2, axis=0, keepdims=True)
        return _relu((x - mu) / jnp.sqrt(var + 1e-5) * g + b)

    x = h_ref[:]
    x = bn_relu(jnp.dot(x, w1[:], preferred_element_type=jnp.float32) + b1[0],
                g1[0], be1[0])
    x = bn_relu(jnp.dot(x, w2[:], preferred_element_type=jnp.float32) + b2[0],
                g2[0], be2[0])
    x = bn_relu(jnp.dot(x, w3[:], preferred_element_type=jnp.float32) + b3[0],
                g3[0], be3[0])
    out_ref[:] = jax.nn.sigmoid(
        jnp.dot(x, wo[:], preferred_element_type=jnp.float32) + bo[0])


def _mlp_head(h, p):
    args = [h]
    for nm in ["W1", "b1", "g1", "be1", "W2", "b2", "g2", "be2",
               "W3", "b3", "g3", "be3", "W_out", "b_out"]:
        v = p[nm]
        args.append(v.reshape(1, -1) if v.ndim == 1 else v)
    return pl.pallas_call(
        _mlp_body,
        out_shape=jax.ShapeDtypeStruct((B, 1), jnp.float32),
    )(*args)


def _segmax_ref(feat, ei, n):
    rows = jnp.take(feat, ei[0], axis=0, mode="clip")
    agg = jax.ops.segment_max(rows, ei[1], num_segments=n)
    return jnp.where(jnp.isfinite(agg), agg, 0.0)


def kernel(finger_feats, seq_feats, MF_feat, BP_feat, CC_feat, params,
           x_dr, x_p, ddi_ei, ppi_ei, mf_sim_ei, bp_sim_ei, cc_sim_ei,
           mf2p_ei, bp2p_ei, cc2p_ei):
    p = params
    h_dr_f = _relu(finger_feats @ p["W_dr_emb"] + p["b_dr_emb"])
    h_p_s = _relu(seq_feats @ p["W_p_emb"] + p["b_p_emb"])
    h_mf = _relu(p["W_mf_emb"] + p["b_mf_emb"])
    h_bp = _relu(p["W_bp_emb"] + p["b_bp_emb"])
    h_cc = _relu(p["W_cc_emb"] + p["b_cc_emb"])

    agg_mf, agg_bp, agg_cc = _sc_segsum([
        (h_mf, mf_sim_ei[0], mf_sim_ei[1], N_MF),
        (h_bp, bp_sim_ei[0], bp_sim_ei[1], N_BP),
        (h_cc, cc_sim_ei[0], cc_sim_ei[1], N_CC),
    ])
    mf_feat = _relu(agg_mf @ p["W_mf_sim"] + p["b_mf_sim"]) + h_mf
    bp_feat = _relu(agg_bp @ p["W_bp_sim"] + p["b_bp_sim"]) + h_bp
    cc_feat = _relu(agg_cc @ p["W_cc_sim"] + p["b_cc_sim"]) + h_cc

    g_mf = mf_feat @ p["W_mf2p"]
    g_bp = bp_feat @ p["W_bp2p"]
    g_cc = cc_feat @ p["W_cc2p"]
    agg_mf2p, agg_bp2p = _sc_segsum([
        (g_mf, mf2p_ei[0], mf2p_ei[1], N_P),
        (g_bp, bp2p_ei[0], bp2p_ei[1], N_P),
    ])
    (agg_cc2p,) = _sc_segsum([
        (g_cc, cc2p_ei[0], cc2p_ei[1], N_P),
    ])
    h_p_go = (_relu(agg_mf2p + p["b_mf2p"]) + _relu(agg_bp2p + p["b_bp2p"])
              + _relu(agg_cc2p + p["b_cc2p"]))

    pool_d1 = _relu(h_dr_f @ p["W_ddi_pool"] + p["b_ddi_pool"])
    pool_p1 = _relu(h_p_s @ p["W_ppi_pool"] + p["b_ppi_pool"])
    agg_d1 = _segmax_ref(pool_d1, ddi_ei, N_DR)
    agg_p1 = _segmax_ref(pool_p1, ppi_ei, N_P)
    h_dr1 = _relu(h_dr_f @ p["W_ddi_self"] + agg_d1 @ p["W_ddi_neigh"]
                  + p["b_ddi"])
    h_p1 = _relu(h_p_s @ p["W_ppi_self"] + agg_p1 @ p["W_ppi_neigh"]
                 + p["b_ppi"])
    pool_d2 = _relu(h_dr1 @ p["W_ddi_pool"] + p["b_ddi_pool"])
    pool_p2 = _relu(h_p1 @ p["W_ppi_pool"] + p["b_ppi_pool"])
    agg_d2 = _segmax_ref(pool_d2, ddi_ei, N_DR)
    agg_p2 = _segmax_ref(pool_p2, ppi_ei, N_P)
    h_dr2 = _relu(h_dr1 @ p["W_ddi_self"] + agg_d2 @ p["W_ddi_neigh"]
                  + p["b_ddi"])
    h_p2 = _relu(h_p1 @ p["W_ppi_self"] + agg_p2 @ p["W_ppi_neigh"]
                 + p["b_ppi"])

    dr_new = jnp.concatenate([h_dr_f, h_dr1, h_dr2], axis=1)
    p_new = jnp.concatenate([h_p_s, h_p1, h_p2, h_p_go], axis=1)
    h = jnp.concatenate([jnp.take(dr_new, x_dr[:, 0], axis=0, mode="clip"),
                         jnp.take(p_new, x_p[:, 0], axis=0, mode="clip")],
                        axis=1)
    return _mlp_head(h, p)
